# baseline (speedup 1.0000x reference)
import jax
import jax.numpy as jnp
from jax import lax
from jax.experimental import pallas as pl
from jax.experimental.pallas import tpu as pltpu
from jax.experimental.pallas import tpu_sc as plsc

N = 10000
D = 128
B = 1024
P = 512
DEG = 32
HALF = DEG // 2 + 1
KPOS = 16 + 1
NW = 32
RPW = B // NW
GPW = P // NW


def _sc_gather_body(t_hbm, l_hbm, rx_hbm, tpm_hbm, o2t_hbm, feat_hbm,
                    nlt_hbm, g_hbm, c_hbm,
                    t_v, l_v, rx_v, nlt_v, gidx_v, grows_v, cidx_v, crows_v,
                    sem):
    wid = lax.axis_index("s") * 2 + lax.axis_index("c")
    base = wid * RPW
    pltpu.sync_copy(t_hbm, t_v)
    pltpu.sync_copy(l_hbm, l_v)
    pltpu.sync_copy(rx_hbm.at[pl.ds(base, RPW)], rx_v)
    lane = lax.iota(jnp.int32, 16)
    zero = lane * 0
    for r in range(RPW):
        for g in range(2):
            idx16 = rx_v[r, pl.ds(g * 16, 16)]
            t16 = plsc.load_gather(t_v, [idx16])
            nl16 = plsc.load_gather(l_v, [t16, zero])
            plsc.store_scatter(nlt_v, [g * 16 + lane, zero + r], nl16)
    pltpu.sync_copy(nlt_v, nlt_hbm.at[:, pl.ds(base, RPW)])
    pltpu.sync_copy(tpm_hbm.at[pl.ds(wid * GPW, GPW)], gidx_v)
    pltpu.async_copy(feat_hbm.at[gidx_v], grows_v, sem).wait()
    pltpu.sync_copy(grows_v, g_hbm.at[pl.ds(wid * GPW, GPW)])
    pltpu.sync_copy(o2t_hbm.at[pl.ds(base, RPW)], cidx_v)
    pltpu.async_copy(feat_hbm.at[cidx_v], crows_v, sem).wait()
    pltpu.sync_copy(crows_v, c_hbm.at[pl.ds(base, RPW)])


def _sc_gather(t_idx, all_logits_flat, rx_list, tpm, o2t, features):
    call = pl.kernel(
        _sc_gather_body,
        out_type=(
            jax.ShapeDtypeStruct((DEG, B), jnp.float32),
            jax.ShapeDtypeStruct((P, D), jnp.float32),
            jax.ShapeDtypeStruct((B, D), jnp.float32),
        ),
        mesh=plsc.VectorSubcoreMesh(core_axis_name="c", subcore_axis_name="s"),
        compiler_params=pltpu.CompilerParams(needs_layout_passes=False,
                                             use_tc_tiling_on_sc=False),
        scratch_types=[
            pltpu.VMEM((N,), jnp.int32),
            pltpu.VMEM((N, 2), jnp.float32),
            pltpu.VMEM((RPW, DEG), jnp.int32),
            pltpu.VMEM((DEG, RPW), jnp.float32),
            pltpu.VMEM((GPW,), jnp.int32),
            pltpu.VMEM((GPW, D), jnp.float32),
            pltpu.VMEM((RPW,), jnp.int32),
            pltpu.VMEM((RPW, D), jnp.float32),
            pltpu.SemaphoreType.DMA,
        ],
    )
    return call(t_idx, all_logits_flat, rx_list, tpm, o2t, features)


def _tc_body(nlt_ref, bcl_ref, lab_ref, tpl_ref, g_ref, c_ref, w_ref, b_ref,
             out_ref):
    c0 = bcl_ref[:, 0:1]
    qcol = tpl_ref[:, 0:1]
    dn0 = (((0,), (0,)), ((), ()))
    hi = lax.Precision.HIGHEST

    ir = lax.broadcasted_iota(jnp.int32, (P, P), 0)
    ic = lax.broadcasted_iota(jnp.int32, (P, P), 1)
    eye_p = (ir == ic).astype(jnp.float32)
    qrow = lax.dot_general(qcol, eye_p, dn0, precision=hi,
                           preferred_element_type=jnp.float32)
    irb = lax.broadcasted_iota(jnp.int32, (B, B), 0)
    icb = lax.broadcasted_iota(jnp.int32, (B, B), 1)
    eye_b = (irb == icb).astype(jnp.float32)
    c0row = lax.dot_general(c0, eye_b, dn0, precision=hi,
                            preferred_element_type=jnp.float32)
    labrow = lab_ref[...].astype(jnp.float32).reshape(1, B)
    labcol = lax.dot_general(eye_b, labrow,
                             (((1,), (1,)), ((), ())), precision=hi,
                             preferred_element_type=jnp.float32)

    cmp = (qrow < qcol) | ((qrow == qcol) & (ic < ir))
    rank = jnp.sum(cmp.astype(jnp.int32), axis=1, keepdims=True)
    onehot = (rank == ic).astype(jnp.float32)
    s_row = lax.dot_general(qcol, onehot, dn0, precision=hi,
                            preferred_element_type=jnp.float32)

    ds = jnp.abs(s_row - c0)
    dsh = pltpu.roll(ds, P - (KPOS - 1), 1)
    col = lax.broadcasted_iota(jnp.int32, (B, P), 1)
    vc = jnp.where(col < P - (KPOS - 1), jnp.maximum(ds, dsh), jnp.inf)
    vstar = jnp.min(vc, axis=1, keepdims=True)
    d2 = jnp.abs(qrow - c0)
    less = d2 < vstar
    eq = d2 == vstar
    c_less = jnp.sum(less.astype(jnp.int32), axis=1, keepdims=True)
    t_b = (KPOS - c_less).astype(jnp.float32)
    lower = (ir < ic).astype(jnp.bfloat16)
    cum = jnp.dot(eq.astype(jnp.bfloat16), lower,
                  preferred_element_type=jnp.float32)
    maskp = less | (eq & (cum < t_b))
    sum_g = jnp.dot(maskp.astype(jnp.float32), g_ref[...],
                    preferred_element_type=jnp.float32)

    dneg = jnp.abs(nlt_ref[...] - c0row)
    bits = lax.bitcast_convert_type(dneg, jnp.int32)
    vstar_n = jnp.zeros((1, B), jnp.int32)
    for kb in range(30, -1, -1):
        t = vstar_n | (1 << kb)
        cnt = jnp.sum((bits < t).astype(jnp.int32), axis=0, keepdims=True)
        vstar_n = jnp.where(cnt < HALF, t, vstar_n)
    c_less_n = jnp.sum((bits < vstar_n).astype(jnp.int32), axis=0,
                       keepdims=True)
    t_bn = HALF - c_less_n
    eqn = bits == vstar_n
    rowi = lax.broadcasted_iota(jnp.int32, (DEG, B), 0)
    istar = jnp.zeros((1, B), jnp.int32)
    for kb in range((DEG - 1).bit_length() - 1, -1, -1):
        t = istar | (1 << kb)
        f = jnp.sum((eqn & (rowi < t)).astype(jnp.int32), axis=0,
                    keepdims=True)
        istar = jnp.where(f < t_bn, t, istar)
    maskn = (bits < vstar_n) | (eqn & (rowi <= istar))
    f_tab = c_ref[0:DEG, :]
    sum_f = lax.dot_general(maskn.astype(jnp.float32), f_tab, dn0,
                            preferred_element_type=jnp.float32)

    agg = jnp.where(labcol == 1.0,
                    (sum_f + sum_g) / (HALF + KPOS),
                    sum_f / HALF)
    w1 = w_ref[:, 0:D]
    w2 = w_ref[:, D:2 * D]
    dn1 = (((1,), (1,)), ((), ()))
    res = (lax.dot_general(c_ref[...], w1, dn1,
                           preferred_element_type=jnp.float32)
           + lax.dot_general(agg, w2, dn1,
                             preferred_element_type=jnp.float32)
           + b_ref[...].reshape(1, D))
    out_ref[...] = jnp.maximum(res, 0.0)


def _tc_call(nlt, bcl, lab, tpl, g_rows, c_rows, w, b):
    return pl.pallas_call(
        _tc_body,
        out_shape=jax.ShapeDtypeStruct((B, D), jnp.float32),
    )(nlt, bcl, lab, tpl, g_rows, c_rows, w, b)


def kernel(features, batch_center_mask, batch_center_labels, train_pos_mask,
           rx_list, batch_center_logits, batch_all_logits, train_pos_logits,
           trainIdx2OrderIdx, orderIdx2trainIdx, avg_half_pos_neigh, W, b):
    nlt, g_rows, c_rows = _sc_gather(
        trainIdx2OrderIdx.astype(jnp.int32),
        batch_all_logits,
        rx_list.astype(jnp.int32),
        train_pos_mask.astype(jnp.int32),
        orderIdx2trainIdx.astype(jnp.int32),
        features,
    )
    return _tc_call(nlt, batch_center_logits,
                    batch_center_labels.astype(jnp.int32), train_pos_logits,
                    g_rows, c_rows, W, b)


# speedup vs baseline: 1.1897x; 1.1897x over previous
import jax
import jax.numpy as jnp
from jax import lax
from jax.experimental import pallas as pl
from jax.experimental.pallas import tpu as pltpu
from jax.experimental.pallas import tpu_sc as plsc

N = 10000
D = 128
B = 1024
P = 512
DEG = 32
HALF = DEG // 2 + 1
KPOS = 16 + 1
NW = 32
RPW = B // NW
GPW = P // NW


def _sc_gather_body(t_hbm, l_hbm, rx_hbm, tpm_hbm, o2t_hbm, feat_hbm,
                    nlt_hbm, g_hbm, c_hbm,
                    t_v, l_v, rx_v, nlt_v, gidx_v, grows_v, cidx_v, crows_v,
                    sem_t, sem_l, sem_rx, sem_gi, sem_ci, sem_g, sem_c):
    wid = lax.axis_index("s") * 2 + lax.axis_index("c")
    base = wid * RPW
    t_cp = pltpu.async_copy(t_hbm, t_v, sem_t)
    l_cp = pltpu.async_copy(l_hbm, l_v, sem_l)
    rx_cp = pltpu.async_copy(rx_hbm.at[pl.ds(base, RPW)], rx_v, sem_rx)
    gi_cp = pltpu.async_copy(tpm_hbm.at[pl.ds(wid * GPW, GPW)], gidx_v,
                             sem_gi)
    ci_cp = pltpu.async_copy(o2t_hbm.at[pl.ds(base, RPW)], cidx_v, sem_ci)
    gi_cp.wait()
    g_cp = pltpu.async_copy(feat_hbm.at[gidx_v], grows_v, sem_g)
    ci_cp.wait()
    c_cp = pltpu.async_copy(feat_hbm.at[cidx_v], crows_v, sem_c)
    t_cp.wait()
    l_cp.wait()
    rx_cp.wait()
    lane = lax.iota(jnp.int32, 16)
    zero = lane * 0
    for r in range(RPW):
        for g in range(2):
            idx16 = rx_v[r, pl.ds(g * 16, 16)]
            t16 = plsc.load_gather(t_v, [idx16])
            nl16 = plsc.load_gather(l_v, [t16 * 2])
            plsc.store_scatter(nlt_v, [g * 16 + lane, zero + r], nl16)
    pltpu.sync_copy(nlt_v, nlt_hbm.at[:, pl.ds(base, RPW)])
    g_cp.wait()
    pltpu.sync_copy(grows_v, g_hbm.at[pl.ds(wid * GPW, GPW)])
    c_cp.wait()
    pltpu.sync_copy(crows_v, c_hbm.at[pl.ds(base, RPW)])


def _sc_gather(t_idx, all_logits_flat, rx_list, tpm, o2t, features):
    call = pl.kernel(
        _sc_gather_body,
        out_type=(
            jax.ShapeDtypeStruct((DEG, B), jnp.float32),
            jax.ShapeDtypeStruct((P, D), jnp.float32),
            jax.ShapeDtypeStruct((B, D), jnp.float32),
        ),
        mesh=plsc.VectorSubcoreMesh(core_axis_name="c", subcore_axis_name="s"),
        compiler_params=pltpu.CompilerParams(needs_layout_passes=False,
                                             use_tc_tiling_on_sc=False),
        scratch_types=[
            pltpu.VMEM((N,), jnp.int32),
            pltpu.VMEM((2 * N,), jnp.float32),
            pltpu.VMEM((RPW, DEG), jnp.int32),
            pltpu.VMEM((DEG, RPW), jnp.float32),
            pltpu.VMEM((GPW,), jnp.int32),
            pltpu.VMEM((GPW, D), jnp.float32),
            pltpu.VMEM((RPW,), jnp.int32),
            pltpu.VMEM((RPW, D), jnp.float32),
        ] + [pltpu.SemaphoreType.DMA] * 7,
    )
    return call(t_idx, all_logits_flat, rx_list, tpm, o2t, features)


def _tc_body(nlt_ref, bcl_ref, c0row_ref, lab_ref, qrow_ref, qcol_ref,
             g_ref, c_ref, w_ref, b_ref, out_ref):
    c0 = bcl_ref[:, 0:1]
    qrow = qrow_ref[...]
    qcol = qcol_ref[...]

    ir = lax.broadcasted_iota(jnp.int32, (P, P), 0)
    ic = lax.broadcasted_iota(jnp.int32, (P, P), 1)
    cmp = (qrow < qcol) | ((qrow == qcol) & (ic < ir))
    rank = jnp.sum(cmp.astype(jnp.int32), axis=1, keepdims=True)
    onehot = (rank == ic).astype(jnp.float32)
    dn0 = (((0,), (0,)), ((), ()))
    s_row = lax.dot_general(qcol, onehot, dn0,
                            precision=lax.Precision.HIGHEST,
                            preferred_element_type=jnp.float32)

    ds = jnp.abs(s_row - c0)
    dsh = pltpu.roll(ds, P - (KPOS - 1), 1)
    col = lax.broadcasted_iota(jnp.int32, (B, P), 1)
    vc = jnp.where(col < P - (KPOS - 1), jnp.maximum(ds, dsh), jnp.inf)
    vstar = jnp.min(vc, axis=1, keepdims=True)
    d2 = jnp.abs(qrow - c0)
    less = d2 < vstar
    eq = d2 == vstar
    c_less = jnp.sum(less.astype(jnp.int32), axis=1, keepdims=True)
    t_b = (KPOS - c_less).astype(jnp.float32)
    lower = (ir < ic).astype(jnp.bfloat16)
    cum = jnp.dot(eq.astype(jnp.bfloat16), lower,
                  preferred_element_type=jnp.float32)
    maskp = less | (eq & (cum < t_b))
    sum_g = jnp.dot(maskp.astype(jnp.float32), g_ref[...],
                    preferred_element_type=jnp.float32)

    dneg = jnp.abs(nlt_ref[...] - c0row_ref[...])
    bits = lax.bitcast_convert_type(dneg, jnp.int32)
    vstar_n = jnp.zeros((1, B), jnp.int32)
    for kb in range(30, -1, -1):
        t = vstar_n | (1 << kb)
        cnt = jnp.sum((bits < t).astype(jnp.int32), axis=0, keepdims=True)
        vstar_n = jnp.where(cnt < HALF, t, vstar_n)
    c_less_n = jnp.sum((bits < vstar_n).astype(jnp.int32), axis=0,
                       keepdims=True)
    t_bn = HALF - c_less_n
    eqn = bits == vstar_n
    rowi = lax.broadcasted_iota(jnp.int32, (DEG, B), 0)
    istar = jnp.zeros((1, B), jnp.int32)
    for kb in range((DEG - 1).bit_length() - 1, -1, -1):
        t = istar | (1 << kb)
        f = jnp.sum((eqn & (rowi < t)).astype(jnp.int32), axis=0,
                    keepdims=True)
        istar = jnp.where(f < t_bn, t, istar)
    maskn = (bits < vstar_n) | (eqn & (rowi <= istar))
    f_tab = c_ref[0:DEG, :]
    sum_f = lax.dot_general(maskn.astype(jnp.float32), f_tab, dn0,
                            preferred_element_type=jnp.float32)

    agg = jnp.where(lab_ref[...] == 1,
                    (sum_f + sum_g) / (HALF + KPOS),
                    sum_f / HALF)
    w1 = w_ref[:, 0:D]
    w2 = w_ref[:, D:2 * D]
    dn1 = (((1,), (1,)), ((), ()))
    res = (lax.dot_general(c_ref[...], w1, dn1,
                           preferred_element_type=jnp.float32)
           + lax.dot_general(agg, w2, dn1,
                             preferred_element_type=jnp.float32)
           + b_ref[...])
    out_ref[...] = jnp.maximum(res, 0.0)


def _tc_call(nlt, bcl, c0row, lab2d, qrow, qcol, g_rows, c_rows, w, b2d):
    return pl.pallas_call(
        _tc_body,
        out_shape=jax.ShapeDtypeStruct((B, D), jnp.float32),
    )(nlt, bcl, c0row, lab2d, qrow, qcol, g_rows, c_rows, w, b2d)


def kernel(features, batch_center_mask, batch_center_labels, train_pos_mask,
           rx_list, batch_center_logits, batch_all_logits, train_pos_logits,
           trainIdx2OrderIdx, orderIdx2trainIdx, avg_half_pos_neigh, W, b):
    nlt, g_rows, c_rows = _sc_gather(
        trainIdx2OrderIdx.astype(jnp.int32),
        batch_all_logits.reshape(2 * N),
        rx_list.astype(jnp.int32),
        train_pos_mask.astype(jnp.int32),
        orderIdx2trainIdx.astype(jnp.int32),
        features,
    )
    qrow = train_pos_logits[:, 0].reshape(1, P)
    qcol = train_pos_logits[:, 0].reshape(P, 1)
    c0row = batch_center_logits[:, 0].reshape(1, B)
    lab2d = batch_center_labels.astype(jnp.int32).reshape(B, 1)
    b2d = b.reshape(1, D)
    return _tc_call(nlt, batch_center_logits, c0row, lab2d, qrow, qcol,
                    g_rows, c_rows, W, b2d)


# speedup vs baseline: 1.4733x; 1.2384x over previous
import jax
import jax.numpy as jnp
from jax import lax
from jax.experimental import pallas as pl
from jax.experimental.pallas import tpu as pltpu
from jax.experimental.pallas import tpu_sc as plsc

N = 10000
D = 128
B = 1024
P = 512
DEG = 32
HALF = DEG // 2 + 1
KPOS = 16 + 1
NW = 32
RPW = B // NW
GPW = P // NW


def _sc_gather_body(t_hbm, l_hbm, rx_hbm, tpm_hbm, o2t_hbm, feat_hbm,
                    nlt_hbm, g_hbm, c_hbm,
                    t_v, l_v, rx_v, nlt_v, gidx_v, grows_v, cidx_v, crows_v,
                    sem_t, sem_l, sem_rx, sem_gi, sem_ci, sem_g, sem_c):
    wid = lax.axis_index("s") * 2 + lax.axis_index("c")
    base = wid * RPW
    t_cp = pltpu.async_copy(t_hbm, t_v, sem_t)
    l_cp = pltpu.async_copy(l_hbm, l_v, sem_l)
    rx_cp = pltpu.async_copy(rx_hbm.at[pl.ds(base, RPW)], rx_v, sem_rx)
    gi_cp = pltpu.async_copy(tpm_hbm.at[pl.ds(wid * GPW, GPW)], gidx_v,
                             sem_gi)
    ci_cp = pltpu.async_copy(o2t_hbm.at[pl.ds(base, RPW)], cidx_v, sem_ci)
    gi_cp.wait()
    g_cp = pltpu.async_copy(feat_hbm.at[gidx_v], grows_v, sem_g)
    ci_cp.wait()
    c_cp = pltpu.async_copy(feat_hbm.at[cidx_v], crows_v, sem_c)
    t_cp.wait()
    l_cp.wait()
    rx_cp.wait()
    lane = lax.iota(jnp.int32, 16)
    zero = lane * 0
    for r in range(RPW):
        for g in range(2):
            idx16 = rx_v[r, pl.ds(g * 16, 16)]
            t16 = plsc.load_gather(t_v, [idx16])
            nl16 = plsc.load_gather(l_v, [t16])
            plsc.store_scatter(nlt_v, [g * 16 + lane, zero + r], nl16)
    pltpu.sync_copy(nlt_v, nlt_hbm.at[:, pl.ds(base, RPW)])
    g_cp.wait()
    pltpu.sync_copy(grows_v, g_hbm.at[pl.ds(wid * GPW, GPW)])
    c_cp.wait()
    pltpu.sync_copy(crows_v, c_hbm.at[pl.ds(base, RPW)])


def _sc_gather(t_idx, all_logits_flat, rx_list, tpm, o2t, features):
    call = pl.kernel(
        _sc_gather_body,
        out_type=(
            jax.ShapeDtypeStruct((DEG, B), jnp.float32),
            jax.ShapeDtypeStruct((P, D), jnp.float32),
            jax.ShapeDtypeStruct((B, D), jnp.float32),
        ),
        mesh=plsc.VectorSubcoreMesh(core_axis_name="c", subcore_axis_name="s"),
        compiler_params=pltpu.CompilerParams(needs_layout_passes=False,
                                             use_tc_tiling_on_sc=False),
        scratch_types=[
            pltpu.VMEM((N,), jnp.int32),
            pltpu.VMEM((N,), jnp.float32),
            pltpu.VMEM((RPW, DEG), jnp.int32),
            pltpu.VMEM((DEG, RPW), jnp.float32),
            pltpu.VMEM((GPW,), jnp.int32),
            pltpu.VMEM((GPW, D), jnp.float32),
            pltpu.VMEM((RPW,), jnp.int32),
            pltpu.VMEM((RPW, D), jnp.float32),
        ] + [pltpu.SemaphoreType.DMA] * 7,
    )
    return call(t_idx, all_logits_flat, rx_list, tpm, o2t, features)


def _tc_maskp_body(bcl_ref, qrow_ref, qcol_ref, maskp_ref):
    c0 = bcl_ref[:, 0:1]
    qrow = qrow_ref[...]
    qcol = qcol_ref[...]

    ir = lax.broadcasted_iota(jnp.int32, (P, P), 0)
    ic = lax.broadcasted_iota(jnp.int32, (P, P), 1)
    cmp = (qrow < qcol) | ((qrow == qcol) & (ic < ir))
    rank = jnp.sum(cmp.astype(jnp.int32), axis=1, keepdims=True)
    onehot = (rank == ic).astype(jnp.float32)
    dn0 = (((0,), (0,)), ((), ()))
    s_row = lax.dot_general(qcol, onehot, dn0,
                            precision=lax.Precision.HIGHEST,
                            preferred_element_type=jnp.float32)

    ds = jnp.abs(s_row - c0)
    dsh = pltpu.roll(ds, P - (KPOS - 1), 1)
    col = lax.broadcasted_iota(jnp.int32, (B, P), 1)
    vc = jnp.where(col < P - (KPOS - 1), jnp.maximum(ds, dsh), jnp.inf)
    vstar = jnp.min(vc, axis=1, keepdims=True)
    d2 = jnp.abs(qrow - c0)
    less = d2 < vstar
    eq = d2 == vstar
    c_less = jnp.sum(less.astype(jnp.int32), axis=1, keepdims=True)
    t_b = (KPOS - c_less).astype(jnp.float32)
    lower = (ir < ic).astype(jnp.bfloat16)
    cum = jnp.dot(eq.astype(jnp.bfloat16), lower,
                  preferred_element_type=jnp.float32)
    maskp_ref[...] = (less | (eq & (cum < t_b))).astype(jnp.float32)


def _tc_body(nlt_ref, c0row_ref, lab_ref, maskp_ref, g_ref, c_ref, w_ref,
             b_ref, out_ref):
    sum_g = jnp.dot(maskp_ref[...], g_ref[...],
                    preferred_element_type=jnp.float32)
    dn0 = (((0,), (0,)), ((), ()))

    dneg = jnp.abs(nlt_ref[...] - c0row_ref[...])
    bits = lax.bitcast_convert_type(dneg, jnp.int32)
    vstar_n = jnp.zeros((1, B), jnp.int32)
    for kb in range(30, -1, -1):
        t = vstar_n | (1 << kb)
        cnt = jnp.sum((bits < t).astype(jnp.int32), axis=0, keepdims=True)
        vstar_n = jnp.where(cnt < HALF, t, vstar_n)
    c_less_n = jnp.sum((bits < vstar_n).astype(jnp.int32), axis=0,
                       keepdims=True)
    t_bn = HALF - c_less_n
    eqn = bits == vstar_n
    rowi = lax.broadcasted_iota(jnp.int32, (DEG, B), 0)
    istar = jnp.zeros((1, B), jnp.int32)
    for kb in range((DEG - 1).bit_length() - 1, -1, -1):
        t = istar | (1 << kb)
        f = jnp.sum((eqn & (rowi < t)).astype(jnp.int32), axis=0,
                    keepdims=True)
        istar = jnp.where(f < t_bn, t, istar)
    maskn = (bits < vstar_n) | (eqn & (rowi <= istar))
    f_tab = c_ref[0:DEG, :]
    sum_f = lax.dot_general(maskn.astype(jnp.float32), f_tab, dn0,
                            preferred_element_type=jnp.float32)

    agg = jnp.where(lab_ref[...] == 1,
                    (sum_f + sum_g) / (HALF + KPOS),
                    sum_f / HALF)
    w1 = w_ref[:, 0:D]
    w2 = w_ref[:, D:2 * D]
    dn1 = (((1,), (1,)), ((), ()))
    res = (lax.dot_general(c_ref[...], w1, dn1,
                           preferred_element_type=jnp.float32)
           + lax.dot_general(agg, w2, dn1,
                             preferred_element_type=jnp.float32)
           + b_ref[...])
    out_ref[...] = jnp.maximum(res, 0.0)


def _tc_call(nlt, bcl, c0row, lab2d, qrow, qcol, g_rows, c_rows, w, b2d):
    maskp = pl.pallas_call(
        _tc_maskp_body,
        out_shape=jax.ShapeDtypeStruct((B, P), jnp.float32),
    )(bcl, qrow, qcol)
    return pl.pallas_call(
        _tc_body,
        out_shape=jax.ShapeDtypeStruct((B, D), jnp.float32),
    )(nlt, c0row, lab2d, maskp, g_rows, c_rows, w, b2d)


def kernel(features, batch_center_mask, batch_center_labels, train_pos_mask,
           rx_list, batch_center_logits, batch_all_logits, train_pos_logits,
           trainIdx2OrderIdx, orderIdx2trainIdx, avg_half_pos_neigh, W, b):
    nlt, g_rows, c_rows = _sc_gather(
        trainIdx2OrderIdx.astype(jnp.int32),
        batch_all_logits[:, 0],
        rx_list.astype(jnp.int32),
        train_pos_mask.astype(jnp.int32),
        orderIdx2trainIdx.astype(jnp.int32),
        features,
    )
    qrow = train_pos_logits[:, 0].reshape(1, P)
    qcol = train_pos_logits[:, 0].reshape(P, 1)
    c0row = batch_center_logits[:, 0].reshape(1, B)
    lab2d = batch_center_labels.astype(jnp.int32).reshape(B, 1)
    b2d = b.reshape(1, D)
    return _tc_call(nlt, batch_center_logits, c0row, lab2d, qrow, qcol,
                    g_rows, c_rows, W, b2d)


# speedup vs baseline: 1.5407x; 1.0457x over previous
import jax
import jax.numpy as jnp
from jax import lax
from jax.experimental import pallas as pl
from jax.experimental.pallas import tpu as pltpu
from jax.experimental.pallas import tpu_sc as plsc

N = 10000
D = 128
B = 1024
P = 512
DEG = 32
HALF = DEG // 2 + 1
KPOS = 16 + 1
NW = 32
RPW = B // NW
GPW = P // NW


def _sc_gather_body(t_hbm, l_hbm, rx_hbm, tpm_hbm, o2t_hbm, feat_hbm,
                    nlt_hbm, g_hbm, c_hbm,
                    t_v, l_v, rx_v, nlt_v, gidx_v, grows_v, cidx_v, crows_v,
                    sem_t, sem_l, sem_rx, sem_gi, sem_ci, sem_g, sem_c):
    wid = lax.axis_index("s") * 2 + lax.axis_index("c")
    base = wid * RPW
    t_cp = pltpu.async_copy(t_hbm, t_v, sem_t)
    l_cp = pltpu.async_copy(l_hbm, l_v, sem_l)
    rx_cp = pltpu.async_copy(rx_hbm.at[pl.ds(base, RPW)], rx_v, sem_rx)
    gi_cp = pltpu.async_copy(tpm_hbm.at[pl.ds(wid * GPW, GPW)], gidx_v,
                             sem_gi)
    ci_cp = pltpu.async_copy(o2t_hbm.at[pl.ds(base, RPW)], cidx_v, sem_ci)
    gi_cp.wait()
    g_cp = pltpu.async_copy(feat_hbm.at[gidx_v], grows_v, sem_g)
    ci_cp.wait()
    c_cp = pltpu.async_copy(feat_hbm.at[cidx_v], crows_v, sem_c)
    t_cp.wait()
    l_cp.wait()
    rx_cp.wait()
    lane = lax.iota(jnp.int32, 16)
    zero = lane * 0
    for r in range(RPW):
        for g in range(2):
            idx16 = rx_v[r, pl.ds(g * 16, 16)]
            t16 = plsc.load_gather(t_v, [idx16])
            nl16 = plsc.load_gather(l_v, [t16])
            plsc.store_scatter(nlt_v, [g * 16 + lane, zero + r], nl16)
    pltpu.sync_copy(nlt_v, nlt_hbm.at[:, pl.ds(base, RPW)])
    g_cp.wait()
    pltpu.sync_copy(grows_v, g_hbm.at[pl.ds(wid * GPW, GPW)])
    c_cp.wait()
    pltpu.sync_copy(crows_v, c_hbm.at[pl.ds(base, RPW)])


def _sc_gather(t_idx, all_logits_flat, rx_list, tpm, o2t, features):
    call = pl.kernel(
        _sc_gather_body,
        out_type=(
            jax.ShapeDtypeStruct((DEG, B), jnp.float32),
            jax.ShapeDtypeStruct((P, D), jnp.float32),
            jax.ShapeDtypeStruct((B, D), jnp.float32),
        ),
        mesh=plsc.VectorSubcoreMesh(core_axis_name="c", subcore_axis_name="s"),
        compiler_params=pltpu.CompilerParams(needs_layout_passes=False,
                                             use_tc_tiling_on_sc=False),
        scratch_types=[
            pltpu.VMEM((N,), jnp.int32),
            pltpu.VMEM((N,), jnp.float32),
            pltpu.VMEM((RPW, DEG), jnp.int32),
            pltpu.VMEM((DEG, RPW), jnp.float32),
            pltpu.VMEM((GPW,), jnp.int32),
            pltpu.VMEM((GPW, D), jnp.float32),
            pltpu.VMEM((RPW,), jnp.int32),
            pltpu.VMEM((RPW, D), jnp.float32),
        ] + [pltpu.SemaphoreType.DMA] * 7,
    )
    return call(t_idx, all_logits_flat, rx_list, tpm, o2t, features)


def _tc_maskp_body(bcl_ref, tpl_ref, lab_ref, maskp_ref, c0row_ref,
                   labcol_ref):
    c0 = bcl_ref[:, 0:1]
    qcol = tpl_ref[:, 0:1]
    dn0 = (((0,), (0,)), ((), ()))
    hi = lax.Precision.HIGHEST

    irb = lax.broadcasted_iota(jnp.int32, (B, B), 0)
    icb = lax.broadcasted_iota(jnp.int32, (B, B), 1)
    eye_b = (irb == icb).astype(jnp.float32)
    c0row_ref[...] = lax.dot_general(c0, eye_b, dn0, precision=hi,
                                     preferred_element_type=jnp.float32)
    labrow = lab_ref[...].astype(jnp.float32).reshape(1, B)
    labcol_ref[...] = lax.dot_general(eye_b, labrow,
                                      (((1,), (1,)), ((), ())), precision=hi,
                                      preferred_element_type=jnp.float32)

    ir = lax.broadcasted_iota(jnp.int32, (P, P), 0)
    ic = lax.broadcasted_iota(jnp.int32, (P, P), 1)
    eye_p = (ir == ic).astype(jnp.float32)
    qrow = lax.dot_general(qcol, eye_p, dn0, precision=hi,
                           preferred_element_type=jnp.float32)
    cmp = (qrow < qcol) | ((qrow == qcol) & (ic < ir))
    rank = jnp.sum(cmp.astype(jnp.int32), axis=1, keepdims=True)
    onehot = (rank == ic).astype(jnp.float32)
    s_row = lax.dot_general(qcol, onehot, dn0, precision=hi,
                            preferred_element_type=jnp.float32)

    ds = jnp.abs(s_row - c0)
    dsh = pltpu.roll(ds, P - (KPOS - 1), 1)
    col = lax.broadcasted_iota(jnp.int32, (B, P), 1)
    vc = jnp.where(col < P - (KPOS - 1), jnp.maximum(ds, dsh), jnp.inf)
    vstar = jnp.min(vc, axis=1, keepdims=True)
    d2 = jnp.abs(qrow - c0)
    less = d2 < vstar
    eq = d2 == vstar
    c_less = jnp.sum(less.astype(jnp.int32), axis=1, keepdims=True)
    t_b = (KPOS - c_less).astype(jnp.float32)
    lower = (ir < ic).astype(jnp.bfloat16)
    cum = jnp.dot(eq.astype(jnp.bfloat16), lower,
                  preferred_element_type=jnp.float32)
    maskp_ref[...] = (less | (eq & (cum < t_b))).astype(jnp.float32)


def _tc_body(nlt_ref, c0row_ref, labcol_ref, maskp_ref, g_ref, c_ref, w_ref,
             b_ref, out_ref):
    sum_g = jnp.dot(maskp_ref[...], g_ref[...],
                    preferred_element_type=jnp.float32)
    dn0 = (((0,), (0,)), ((), ()))

    dneg = jnp.abs(nlt_ref[...] - c0row_ref[...])
    bits = lax.bitcast_convert_type(dneg, jnp.int32)
    vstar_n = jnp.zeros((1, B), jnp.int32)
    for kb in range(30, -1, -1):
        t = vstar_n | (1 << kb)
        cnt = jnp.sum((bits < t).astype(jnp.int32), axis=0, keepdims=True)
        vstar_n = jnp.where(cnt < HALF, t, vstar_n)
    c_less_n = jnp.sum((bits < vstar_n).astype(jnp.int32), axis=0,
                       keepdims=True)
    t_bn = HALF - c_less_n
    eqn = bits == vstar_n
    rowi = lax.broadcasted_iota(jnp.int32, (DEG, B), 0)
    istar = jnp.zeros((1, B), jnp.int32)
    for kb in range((DEG - 1).bit_length() - 1, -1, -1):
        t = istar | (1 << kb)
        f = jnp.sum((eqn & (rowi < t)).astype(jnp.int32), axis=0,
                    keepdims=True)
        istar = jnp.where(f < t_bn, t, istar)
    maskn = (bits < vstar_n) | (eqn & (rowi <= istar))
    f_tab = c_ref[0:DEG, :]
    sum_f = lax.dot_general(maskn.astype(jnp.float32), f_tab, dn0,
                            preferred_element_type=jnp.float32)

    agg = jnp.where(labcol_ref[...] == 1.0,
                    (sum_f + sum_g) / (HALF + KPOS),
                    sum_f / HALF)
    w1 = w_ref[:, 0:D]
    w2 = w_ref[:, D:2 * D]
    dn1 = (((1,), (1,)), ((), ()))
    res = (lax.dot_general(c_ref[...], w1, dn1,
                           preferred_element_type=jnp.float32)
           + lax.dot_general(agg, w2, dn1,
                             preferred_element_type=jnp.float32)
           + b_ref[...].reshape(1, D))
    out_ref[...] = jnp.maximum(res, 0.0)


def _tc_call(nlt, bcl, lab, tpl, g_rows, c_rows, w, b):
    maskp, c0row, labcol = pl.pallas_call(
        _tc_maskp_body,
        out_shape=(jax.ShapeDtypeStruct((B, P), jnp.float32),
                   jax.ShapeDtypeStruct((1, B), jnp.float32),
                   jax.ShapeDtypeStruct((B, 1), jnp.float32)),
    )(bcl, tpl, lab)
    return pl.pallas_call(
        _tc_body,
        out_shape=jax.ShapeDtypeStruct((B, D), jnp.float32),
    )(nlt, c0row, labcol, maskp, g_rows, c_rows, w, b)


def kernel(features, batch_center_mask, batch_center_labels, train_pos_mask,
           rx_list, batch_center_logits, batch_all_logits, train_pos_logits,
           trainIdx2OrderIdx, orderIdx2trainIdx, avg_half_pos_neigh, W, b):
    nlt, g_rows, c_rows = _sc_gather(
        trainIdx2OrderIdx.astype(jnp.int32),
        batch_all_logits[:, 0],
        rx_list.astype(jnp.int32),
        train_pos_mask.astype(jnp.int32),
        orderIdx2trainIdx.astype(jnp.int32),
        features,
    )
    return _tc_call(nlt, batch_center_logits,
                    batch_center_labels.astype(jnp.int32), train_pos_logits,
                    g_rows, c_rows, W, b)
